# Initial kernel scaffold; baseline (speedup 1.0000x reference)
import jax
import jax.numpy as jnp
from jax.experimental import pallas as pl


def kernel(x, edge_index, batch, W_self_w, W_self_b, W_neigh_w, W_neigh_b, W_fp_w):
    raise NotImplementedError("write your pallas kernel here")


# speedup vs baseline: 2.5004x; 2.5004x over previous
import jax
import jax.numpy as jnp
from jax import lax
from jax.experimental import pallas as pl
from jax.experimental.pallas import tpu as pltpu
from jax.experimental.pallas import tpu_sc as plsc

_NUM_LAYERS = 3
_G = 64
_N = 10000
_E = 320000
_D = 128

_NC = 2
_NS = 16
_NW = _NC * _NS

_BAND = 320
_N_PAD = _BAND * _NW
_CORE_ROWS = _BAND * _NS
_ACC_ROWS = _CORE_ROWS + 8

_CHUNK = 128
_LMAX = 12288
_SCAN = 512
_NSCAN = _E // _SCAN


def _prepass_body(row_hbm, col_hbm, mc_hbm, md_hbm, cnt_hbm,
                  rbuf, cbuf, mcol, mdst, cntv):
    c = lax.axis_index("c")
    s = lax.axis_index("s")
    w = c * _NS + s
    core_base = c * _CORE_ROWS
    lo = core_base + s * _BAND
    hi = lo + _BAND
    trash = _CORE_ROWS + (s % 8)
    dummy_col = 10000 + s * 15

    def init(i, carry):
        mcol[pl.ds(i * 16, 16)] = jnp.full((16,), dummy_col, jnp.int32)
        mdst[pl.ds(i * 16, 16)] = jnp.full((16,), trash, jnp.int32)
        return carry

    lax.fori_loop(0, _LMAX // 16, init, 0)

    def chunk(jc, nm):
        pltpu.sync_copy(row_hbm.at[pl.ds(jc * _SCAN, _SCAN)], rbuf)
        pltpu.sync_copy(col_hbm.at[pl.ds(jc * _SCAN, _SCAN)], cbuf)

        def inner(i, nm):
            dv = rbuf[pl.ds(i * 16, 16)]
            cv = cbuf[pl.ds(i * 16, 16)]
            m = (dv >= lo) & (dv < hi)
            off = jnp.minimum(nm, _LMAX - 16)
            ranks = plsc.cumsum(m.astype(jnp.int32)) - 1
            pos = jnp.where(m, off + ranks, _LMAX)
            plsc.store_scatter(mcol, [pos], cv)
            plsc.store_scatter(mdst, [pos], dv - core_base)
            return nm + (jnp.max(ranks) + 1)

        return lax.fori_loop(0, _SCAN // 16, inner, nm)

    nm = lax.fori_loop(0, _NSCAN, chunk, 0)

    pltpu.sync_copy(mcol.at[pl.ds(0, _LMAX)], mc_hbm.at[pl.ds(w * _LMAX, _LMAX)])
    pltpu.sync_copy(mdst.at[pl.ds(0, _LMAX)], md_hbm.at[pl.ds(w * _LMAX, _LMAX)])
    cntv[...] = jnp.full((16,), nm, jnp.int32)
    pltpu.sync_copy(cntv, cnt_hbm.at[pl.ds(w * 16, 16)])


def _prepass(row, col):
    fn = pl.kernel(
        _prepass_body,
        out_type=(jax.ShapeDtypeStruct((_NW * _LMAX,), jnp.int32),
                  jax.ShapeDtypeStruct((_NW * _LMAX,), jnp.int32),
                  jax.ShapeDtypeStruct((_NW * 16,), jnp.int32)),
        mesh=plsc.VectorSubcoreMesh(core_axis_name="c", subcore_axis_name="s"),
        scratch_types=[
            pltpu.VMEM((_SCAN,), jnp.int32),
            pltpu.VMEM((_SCAN,), jnp.int32),
            pltpu.VMEM((_LMAX + 16,), jnp.int32),
            pltpu.VMEM((_LMAX + 16,), jnp.int32),
            pltpu.VMEM((16,), jnp.int32),
        ],
        compiler_params=pltpu.CompilerParams(needs_layout_passes=False),
    )
    return fn(row, col)


def _neigh_body(x_hbm, mc_hbm, md_hbm, cnt_hbm, zeros_hbm, out_hbm,
                colv, dstv, rowsv, cbuf, accum, sem):
    c = lax.axis_index("c")
    s = lax.axis_index("s")
    w = c * _NS + s

    pltpu.sync_copy(zeros_hbm.at[pl.ds(s * _BAND, _BAND)],
                    accum.at[pl.ds(s * _BAND, _BAND)])

    @pl.when(s == 0)
    def _():
        pltpu.sync_copy(zeros_hbm.at[pl.ds(_CORE_ROWS, 8)],
                        accum.at[pl.ds(_CORE_ROWS, 8)])

    plsc.subcore_barrier()

    pltpu.sync_copy(cnt_hbm.at[pl.ds(w * 16, 16)], cbuf)
    cnt = jnp.max(cbuf[...])
    nch = (cnt + _CHUNK - 1) // _CHUNK

    def it(g, carry):
        off = w * _LMAX + g * _CHUNK
        pltpu.sync_copy(mc_hbm.at[pl.ds(off, _CHUNK)], colv)
        pltpu.async_copy(x_hbm.at[colv], rowsv, sem).wait()
        pltpu.sync_copy(md_hbm.at[pl.ds(off, _CHUNK)], dstv)
        pltpu.sync_copy(rowsv, accum.at[dstv], add=True)
        return carry

    lax.fori_loop(0, nch, it, 0)
    plsc.subcore_barrier()

    pltpu.sync_copy(
        accum.at[pl.ds(s * _BAND, _BAND)],
        out_hbm.at[pl.ds(c * _CORE_ROWS + s * _BAND, _BAND)])


def _neigh_call(x_pad, mc, md, cnt, zeros):
    fn = pl.kernel(
        _neigh_body,
        out_type=jax.ShapeDtypeStruct((_N_PAD, _D), jnp.float32),
        mesh=plsc.VectorSubcoreMesh(core_axis_name="c", subcore_axis_name="s"),
        scratch_types=[
            pltpu.VMEM((_CHUNK,), jnp.int32),
            pltpu.VMEM((_CHUNK,), jnp.int32),
            pltpu.VMEM((_CHUNK, _D), jnp.float32),
            pltpu.VMEM((16,), jnp.int32),
            pltpu.VMEM_SHARED((_ACC_ROWS, _D), jnp.float32),
            pltpu.SemaphoreType.DMA,
        ],
        compiler_params=pltpu.CompilerParams(needs_layout_passes=False),
    )
    return fn(x_pad, mc, md, cnt, zeros)


_BLK = 1280


def _tc_layer_body(x_ref, n_ref, ws_ref, bs_ref, wn_ref, bn_ref,
                   wfp_ref, seg_ref, fp_in_ref, xo_ref, fp_ref):
    i = pl.program_id(0)
    x = x_ref[...]
    n = n_ref[...]
    h = jnp.tanh(
        lax.dot_general(x, ws_ref[...], (((1,), (1,)), ((), ())),
                        preferred_element_type=jnp.float32)
        + bs_ref[...]
        + lax.dot_general(n, wn_ref[...], (((1,), (1,)), ((), ())),
                          preferred_element_type=jnp.float32)
        + bn_ref[...])
    rows = lax.broadcasted_iota(jnp.int32, (_BLK, 1), 0) + i * _BLK
    h = jnp.where(rows < _N, h, 0.0)
    xo_ref[...] = h

    logits = lax.dot_general(h, wfp_ref[...], (((1,), (1,)), ((), ())),
                             preferred_element_type=jnp.float32)
    m = jnp.max(logits, axis=1, keepdims=True)
    e = jnp.exp(logits - m)
    p = e / jnp.sum(e, axis=1, keepdims=True)

    seg = seg_ref[...]
    onehot = (seg == lax.broadcasted_iota(jnp.int32, (_BLK, _G), 1)
              ).astype(jnp.float32)
    delta = lax.dot_general(onehot, p, (((0,), (0,)), ((), ())),
                            preferred_element_type=jnp.float32)

    @pl.when(i == 0)
    def _():
        fp_ref[...] = fp_in_ref[...] + delta

    @pl.when(i != 0)
    def _():
        fp_ref[...] = fp_ref[...] + delta


def _tc_layer(x_pad, n, ws, bs, wn, bn, wfp, seg, fp_in):
    nblk = _N_PAD // _BLK
    row_spec = pl.BlockSpec((_BLK, _D), lambda i: (i, 0))
    full_spec = pl.BlockSpec((_D, _D), lambda i: (0, 0))
    bias_spec = pl.BlockSpec((1, _D), lambda i: (0, 0))
    seg_spec = pl.BlockSpec((_BLK, 1), lambda i: (i, 0))
    fp_spec = pl.BlockSpec((_G, _D), lambda i: (0, 0))
    return pl.pallas_call(
        _tc_layer_body,
        grid=(nblk,),
        in_specs=[row_spec, row_spec, full_spec, bias_spec,
                  full_spec, bias_spec, full_spec, seg_spec, fp_spec],
        out_specs=[row_spec, fp_spec],
        out_shape=[jax.ShapeDtypeStruct((_N_PAD, _D), jnp.float32),
                   jax.ShapeDtypeStruct((_G, _D), jnp.float32)],
    )(x_pad, n, ws, bs, wn, bn, wfp, seg, fp_in)


def kernel(x, edge_index, batch, W_self_w, W_self_b, W_neigh_w, W_neigh_b,
           W_fp_w):
    x = x.astype(jnp.float32)
    row = edge_index[0].astype(jnp.int32)
    col = edge_index[1].astype(jnp.int32)

    mc, md, cnt = _prepass(row, col)

    x_pad = jnp.concatenate(
        [x, jnp.zeros((_N_PAD - _N, _D), jnp.float32)], axis=0)
    seg = jnp.concatenate(
        [batch.astype(jnp.int32),
         jnp.full((_N_PAD - _N,), _G, jnp.int32)]).reshape(_N_PAD, 1)
    zeros = jnp.zeros((_ACC_ROWS, _D), jnp.float32)

    fp = jnp.zeros((_G, _D), jnp.float32)
    for layer in range(_NUM_LAYERS):
        nb = _neigh_call(x_pad, mc, md, cnt, zeros)
        x_pad, fp = _tc_layer(
            x_pad, nb,
            W_self_w[layer], W_self_b[layer].reshape(1, _D),
            W_neigh_w[layer], W_neigh_b[layer].reshape(1, _D),
            W_fp_w[layer], seg, fp)
    return fp


# speedup vs baseline: 3.1263x; 1.2503x over previous
import jax
import jax.numpy as jnp
from jax import lax
from jax.experimental import pallas as pl
from jax.experimental.pallas import tpu as pltpu
from jax.experimental.pallas import tpu_sc as plsc

_NUM_LAYERS = 3
_G = 64
_N = 10000
_E = 320000
_D = 128

_NC = 2
_NS = 16
_NW = _NC * _NS

_BAND = 320
_N_PAD = _BAND * _NW
_CORE_ROWS = _BAND * _NS
_ACC_ROWS = _CORE_ROWS + 8

_CHUNK = 128
_LMAX = 12288
_SCAN = 512
_NSCAN = _E // _SCAN


def _prepass_body(row_hbm, col_hbm, mc_hbm, md_hbm, cnt_hbm,
                  rbuf, cbuf, mcol, mdst, cntv):
    c = lax.axis_index("c")
    s = lax.axis_index("s")
    w = c * _NS + s
    core_base = c * _CORE_ROWS
    lo = core_base + s * _BAND
    hi = lo + _BAND
    trash = _CORE_ROWS + (s % 8)
    dummy_col = 10000 + s * 15

    def init(i, carry):
        mcol[pl.ds(i * 16, 16)] = jnp.full((16,), dummy_col, jnp.int32)
        mdst[pl.ds(i * 16, 16)] = jnp.full((16,), trash, jnp.int32)
        return carry

    lax.fori_loop(0, _LMAX // 16, init, 0)

    def chunk(jc, nm):
        pltpu.sync_copy(row_hbm.at[pl.ds(jc * _SCAN, _SCAN)], rbuf)
        pltpu.sync_copy(col_hbm.at[pl.ds(jc * _SCAN, _SCAN)], cbuf)

        def inner(i, nm):
            dv = rbuf[pl.ds(i * 16, 16)]
            cv = cbuf[pl.ds(i * 16, 16)]
            m = (dv >= lo) & (dv < hi)
            base = jnp.minimum(nm, _LMAX - 16)
            ranks = plsc.cumsum(m.astype(jnp.int32)) - 1
            pos = jnp.where(m, base + ranks, _LMAX)
            plsc.store_scatter(mcol, [pos], cv)
            plsc.store_scatter(mdst, [pos], dv - core_base)
            return nm + plsc.all_reduce_population_count(m)

        return lax.fori_loop(0, _SCAN // 16, inner, nm, unroll=2)

    nm = lax.fori_loop(0, _NSCAN, chunk, jnp.zeros((16,), jnp.int32))

    pltpu.sync_copy(mcol.at[pl.ds(0, _LMAX)], mc_hbm.at[pl.ds(w * _LMAX, _LMAX)])
    pltpu.sync_copy(mdst.at[pl.ds(0, _LMAX)], md_hbm.at[pl.ds(w * _LMAX, _LMAX)])
    cntv[...] = nm
    pltpu.sync_copy(cntv, cnt_hbm.at[pl.ds(w * 16, 16)])


def _prepass(row, col):
    fn = pl.kernel(
        _prepass_body,
        out_type=(jax.ShapeDtypeStruct((_NW * _LMAX,), jnp.int32),
                  jax.ShapeDtypeStruct((_NW * _LMAX,), jnp.int32),
                  jax.ShapeDtypeStruct((_NW * 16,), jnp.int32)),
        mesh=plsc.VectorSubcoreMesh(core_axis_name="c", subcore_axis_name="s"),
        scratch_types=[
            pltpu.VMEM((_SCAN,), jnp.int32),
            pltpu.VMEM((_SCAN,), jnp.int32),
            pltpu.VMEM((_LMAX + 16,), jnp.int32),
            pltpu.VMEM((_LMAX + 16,), jnp.int32),
            pltpu.VMEM((16,), jnp.int32),
        ],
        compiler_params=pltpu.CompilerParams(needs_layout_passes=False),
    )
    return fn(row, col)


def _neigh_body(x_hbm, mc_hbm, md_hbm, cnt_hbm, zeros_hbm, out_hbm,
                colv0, colv1, dstv, rows0, rows1, cbuf, accum, sem0, sem1):
    c = lax.axis_index("c")
    s = lax.axis_index("s")
    w = c * _NS + s

    pltpu.sync_copy(zeros_hbm.at[pl.ds(s * _BAND, _BAND)],
                    accum.at[pl.ds(s * _BAND, _BAND)])

    @pl.when(s == 0)
    def _():
        pltpu.sync_copy(zeros_hbm.at[pl.ds(_CORE_ROWS, 8)],
                        accum.at[pl.ds(_CORE_ROWS, 8)])

    plsc.subcore_barrier()

    pltpu.sync_copy(cnt_hbm.at[pl.ds(w * 16, 16)], cbuf)
    cnt = jnp.max(cbuf[...])
    nch = (cnt + _CHUNK - 1) // _CHUNK
    base = w * _LMAX

    @pl.when(nch > 0)
    def _():
        pltpu.sync_copy(mc_hbm.at[pl.ds(base, _CHUNK)], colv0)
        pltpu.async_copy(x_hbm.at[colv0], rows0, sem0)

    def it(t, carry):
        g0 = 2 * t
        g1 = g0 + 1

        @pl.when(g1 < nch)
        def _():
            pltpu.sync_copy(mc_hbm.at[pl.ds(base + g1 * _CHUNK, _CHUNK)], colv1)
            pltpu.async_copy(x_hbm.at[colv1], rows1, sem1)

        pltpu.sync_copy(md_hbm.at[pl.ds(base + g0 * _CHUNK, _CHUNK)], dstv)
        pltpu.make_async_copy(x_hbm.at[colv0], rows0, sem0).wait()
        pltpu.sync_copy(rows0, accum.at[dstv], add=True)

        @pl.when(g0 + 2 < nch)
        def _():
            pltpu.sync_copy(mc_hbm.at[pl.ds(base + (g0 + 2) * _CHUNK, _CHUNK)],
                            colv0)
            pltpu.async_copy(x_hbm.at[colv0], rows0, sem0)

        @pl.when(g1 < nch)
        def _():
            pltpu.sync_copy(md_hbm.at[pl.ds(base + g1 * _CHUNK, _CHUNK)], dstv)
            pltpu.make_async_copy(x_hbm.at[colv1], rows1, sem1).wait()
            pltpu.sync_copy(rows1, accum.at[dstv], add=True)

        return carry

    lax.fori_loop(0, (nch + 1) // 2, it, 0)
    plsc.subcore_barrier()

    pltpu.sync_copy(
        accum.at[pl.ds(s * _BAND, _BAND)],
        out_hbm.at[pl.ds(c * _CORE_ROWS + s * _BAND, _BAND)])


def _neigh_call(x_pad, mc, md, cnt, zeros):
    fn = pl.kernel(
        _neigh_body,
        out_type=jax.ShapeDtypeStruct((_N_PAD, _D), jnp.float32),
        mesh=plsc.VectorSubcoreMesh(core_axis_name="c", subcore_axis_name="s"),
        scratch_types=[
            pltpu.VMEM((_CHUNK,), jnp.int32),
            pltpu.VMEM((_CHUNK,), jnp.int32),
            pltpu.VMEM((_CHUNK,), jnp.int32),
            pltpu.VMEM((_CHUNK, _D), jnp.float32),
            pltpu.VMEM((_CHUNK, _D), jnp.float32),
            pltpu.VMEM((16,), jnp.int32),
            pltpu.VMEM_SHARED((_ACC_ROWS, _D), jnp.float32),
            pltpu.SemaphoreType.DMA,
            pltpu.SemaphoreType.DMA,
        ],
        compiler_params=pltpu.CompilerParams(needs_layout_passes=False),
    )
    return fn(x_pad, mc, md, cnt, zeros)


_BLK = 1280


def _tc_layer_body(x_ref, n_ref, ws_ref, bs_ref, wn_ref, bn_ref,
                   wfp_ref, seg_ref, fp_in_ref, xo_ref, fp_ref):
    i = pl.program_id(0)
    x = x_ref[...]
    n = n_ref[...]
    h = jnp.tanh(
        lax.dot_general(x, ws_ref[...], (((1,), (1,)), ((), ())),
                        preferred_element_type=jnp.float32)
        + bs_ref[...]
        + lax.dot_general(n, wn_ref[...], (((1,), (1,)), ((), ())),
                          preferred_element_type=jnp.float32)
        + bn_ref[...])
    rows = lax.broadcasted_iota(jnp.int32, (_BLK, 1), 0) + i * _BLK
    h = jnp.where(rows < _N, h, 0.0)
    xo_ref[...] = h

    logits = lax.dot_general(h, wfp_ref[...], (((1,), (1,)), ((), ())),
                             preferred_element_type=jnp.float32)
    m = jnp.max(logits, axis=1, keepdims=True)
    e = jnp.exp(logits - m)
    p = e / jnp.sum(e, axis=1, keepdims=True)

    seg = seg_ref[...]
    onehot = (seg == lax.broadcasted_iota(jnp.int32, (_BLK, _G), 1)
              ).astype(jnp.float32)
    delta = lax.dot_general(onehot, p, (((0,), (0,)), ((), ())),
                            preferred_element_type=jnp.float32)

    @pl.when(i == 0)
    def _():
        fp_ref[...] = fp_in_ref[...] + delta

    @pl.when(i != 0)
    def _():
        fp_ref[...] = fp_ref[...] + delta


def _tc_layer(x_pad, n, ws, bs, wn, bn, wfp, seg, fp_in):
    nblk = _N_PAD // _BLK
    row_spec = pl.BlockSpec((_BLK, _D), lambda i: (i, 0))
    full_spec = pl.BlockSpec((_D, _D), lambda i: (0, 0))
    bias_spec = pl.BlockSpec((1, _D), lambda i: (0, 0))
    seg_spec = pl.BlockSpec((_BLK, 1), lambda i: (i, 0))
    fp_spec = pl.BlockSpec((_G, _D), lambda i: (0, 0))
    return pl.pallas_call(
        _tc_layer_body,
        grid=(nblk,),
        in_specs=[row_spec, row_spec, full_spec, bias_spec,
                  full_spec, bias_spec, full_spec, seg_spec, fp_spec],
        out_specs=[row_spec, fp_spec],
        out_shape=[jax.ShapeDtypeStruct((_N_PAD, _D), jnp.float32),
                   jax.ShapeDtypeStruct((_G, _D), jnp.float32)],
    )(x_pad, n, ws, bs, wn, bn, wfp, seg, fp_in)


def kernel(x, edge_index, batch, W_self_w, W_self_b, W_neigh_w, W_neigh_b,
           W_fp_w):
    x = x.astype(jnp.float32)
    row = edge_index[0].astype(jnp.int32)
    col = edge_index[1].astype(jnp.int32)

    mc, md, cnt = _prepass(row, col)

    x_pad = jnp.concatenate(
        [x, jnp.zeros((_N_PAD - _N, _D), jnp.float32)], axis=0)
    seg = jnp.concatenate(
        [batch.astype(jnp.int32),
         jnp.full((_N_PAD - _N,), _G, jnp.int32)]).reshape(_N_PAD, 1)
    zeros = jnp.zeros((_ACC_ROWS, _D), jnp.float32)

    fp = jnp.zeros((_G, _D), jnp.float32)
    for layer in range(_NUM_LAYERS):
        nb = _neigh_call(x_pad, mc, md, cnt, zeros)
        x_pad, fp = _tc_layer(
            x_pad, nb,
            W_self_w[layer], W_self_b[layer].reshape(1, _D),
            W_neigh_w[layer], W_neigh_b[layer].reshape(1, _D),
            W_fp_w[layer], seg, fp)
    return fp


# speedup vs baseline: 5.6438x; 1.8053x over previous
import jax
import jax.numpy as jnp
from jax import lax
from jax.experimental import pallas as pl
from jax.experimental.pallas import tpu as pltpu
from jax.experimental.pallas import tpu_sc as plsc

_NUM_LAYERS = 3
_G = 64
_N = 10000
_E = 320000
_D = 128

_NC = 2
_NS = 16
_NW = _NC * _NS

_BAND = 320
_N_PAD = _BAND * _NW
_CORE_ROWS = _BAND * _NS
_ACC_ROWS = _CORE_ROWS + 8

_CHUNK = 128
_LMAX = 12288
_SCAN = 2560
_NSCAN = _E // _SCAN


def _prepass_body(row_hbm, col_hbm, mc_hbm, md_hbm, cnt_hbm,
                  rbuf0, cbuf0, rbuf1, cbuf1, mcol, mdst, cntv, sem0, sem1):
    c = lax.axis_index("c")
    s = lax.axis_index("s")
    w = c * _NS + s
    core_base = c * _CORE_ROWS
    lo = core_base + s * _BAND
    hi = lo + _BAND
    trash = _CORE_ROWS + (s % 8)
    dummy_col = 10000 + s * 15

    def init(i, carry):
        mcol[pl.ds(i * 16, 16)] = jnp.full((16,), dummy_col, jnp.int32)
        mdst[pl.ds(i * 16, 16)] = jnp.full((16,), trash, jnp.int32)
        return carry

    lax.fori_loop(0, _LMAX // 16, init, 0)

    def start(jc, rbuf, cbuf, sem):
        pltpu.async_copy(row_hbm.at[pl.ds(jc * _SCAN, _SCAN)], rbuf, sem)
        pltpu.async_copy(col_hbm.at[pl.ds(jc * _SCAN, _SCAN)], cbuf, sem)

    def wait(jc, rbuf, cbuf, sem):
        pltpu.make_async_copy(row_hbm.at[pl.ds(jc * _SCAN, _SCAN)], rbuf, sem).wait()
        pltpu.make_async_copy(col_hbm.at[pl.ds(jc * _SCAN, _SCAN)], cbuf, sem).wait()

    def scan(rbuf, cbuf, nm):
        def inner(i, nm):
            dv = rbuf[pl.ds(i * 16, 16)]
            cv = cbuf[pl.ds(i * 16, 16)]
            m = (dv >= lo) & (dv < hi)
            base = jnp.minimum(nm, _LMAX - 16)
            ranks = plsc.cumsum(m.astype(jnp.int32)) - 1
            pos = jnp.where(m, base + ranks, _LMAX)
            plsc.store_scatter(mcol, [pos], cv)
            plsc.store_scatter(mdst, [pos], dv - core_base)
            return nm + plsc.all_reduce_population_count(m)

        return lax.fori_loop(0, _SCAN // 16, inner, nm, unroll=4)

    start(0, rbuf0, cbuf0, sem0)

    def chunk2(t, nm):
        j0 = 2 * t
        j1 = j0 + 1

        @pl.when(j1 < _NSCAN)
        def _():
            start(j1, rbuf1, cbuf1, sem1)

        wait(j0, rbuf0, cbuf0, sem0)
        nm = scan(rbuf0, cbuf0, nm)

        @pl.when(j0 + 2 < _NSCAN)
        def _():
            start(j0 + 2, rbuf0, cbuf0, sem0)

        def do1(nm):
            wait(j1, rbuf1, cbuf1, sem1)
            return scan(rbuf1, cbuf1, nm)

        nm = lax.cond(j1 < _NSCAN, do1, lambda nm: nm, nm)
        return nm

    nm = lax.fori_loop(0, (_NSCAN + 1) // 2, chunk2,
                       jnp.zeros((16,), jnp.int32))

    pltpu.sync_copy(mcol.at[pl.ds(0, _LMAX)], mc_hbm.at[pl.ds(w * _LMAX, _LMAX)])
    pltpu.sync_copy(mdst.at[pl.ds(0, _LMAX)], md_hbm.at[pl.ds(w * _LMAX, _LMAX)])
    cntv[...] = nm
    pltpu.sync_copy(cntv, cnt_hbm.at[pl.ds(w * 16, 16)])


def _prepass(row, col):
    fn = pl.kernel(
        _prepass_body,
        out_type=(jax.ShapeDtypeStruct((_NW * _LMAX,), jnp.int32),
                  jax.ShapeDtypeStruct((_NW * _LMAX,), jnp.int32),
                  jax.ShapeDtypeStruct((_NW * 16,), jnp.int32)),
        mesh=plsc.VectorSubcoreMesh(core_axis_name="c", subcore_axis_name="s"),
        scratch_types=[
            pltpu.VMEM((_SCAN,), jnp.int32),
            pltpu.VMEM((_SCAN,), jnp.int32),
            pltpu.VMEM((_SCAN,), jnp.int32),
            pltpu.VMEM((_SCAN,), jnp.int32),
            pltpu.VMEM((_LMAX + 16,), jnp.int32),
            pltpu.VMEM((_LMAX + 16,), jnp.int32),
            pltpu.VMEM((16,), jnp.int32),
            pltpu.SemaphoreType.DMA,
            pltpu.SemaphoreType.DMA,
        ],
        compiler_params=pltpu.CompilerParams(needs_layout_passes=False),
    )
    return fn(row, col)


def _neigh_body(x_hbm, mc_hbm, md_hbm, cnt_hbm, zeros_hbm, out_hbm,
                colv0, colv1, dstv, rows0, rows1, cbuf, accum, sem0, sem1):
    c = lax.axis_index("c")
    s = lax.axis_index("s")
    w = c * _NS + s

    pltpu.sync_copy(zeros_hbm.at[pl.ds(s * _BAND, _BAND)],
                    accum.at[pl.ds(s * _BAND, _BAND)])

    @pl.when(s == 0)
    def _():
        pltpu.sync_copy(zeros_hbm.at[pl.ds(_CORE_ROWS, 8)],
                        accum.at[pl.ds(_CORE_ROWS, 8)])

    plsc.subcore_barrier()

    pltpu.sync_copy(cnt_hbm.at[pl.ds(w * 16, 16)], cbuf)
    cnt = jnp.max(cbuf[...])
    nch = (cnt + _CHUNK - 1) // _CHUNK
    base = w * _LMAX

    @pl.when(nch > 0)
    def _():
        pltpu.sync_copy(mc_hbm.at[pl.ds(base, _CHUNK)], colv0)
        pltpu.async_copy(x_hbm.at[colv0], rows0, sem0)

    def it(t, carry):
        g0 = 2 * t
        g1 = g0 + 1

        @pl.when(g1 < nch)
        def _():
            pltpu.sync_copy(mc_hbm.at[pl.ds(base + g1 * _CHUNK, _CHUNK)], colv1)
            pltpu.async_copy(x_hbm.at[colv1], rows1, sem1)

        pltpu.sync_copy(md_hbm.at[pl.ds(base + g0 * _CHUNK, _CHUNK)], dstv)
        pltpu.make_async_copy(x_hbm.at[colv0], rows0, sem0).wait()
        pltpu.sync_copy(rows0, accum.at[dstv], add=True)

        @pl.when(g0 + 2 < nch)
        def _():
            pltpu.sync_copy(mc_hbm.at[pl.ds(base + (g0 + 2) * _CHUNK, _CHUNK)],
                            colv0)
            pltpu.async_copy(x_hbm.at[colv0], rows0, sem0)

        @pl.when(g1 < nch)
        def _():
            pltpu.sync_copy(md_hbm.at[pl.ds(base + g1 * _CHUNK, _CHUNK)], dstv)
            pltpu.make_async_copy(x_hbm.at[colv1], rows1, sem1).wait()
            pltpu.sync_copy(rows1, accum.at[dstv], add=True)

        return carry

    lax.fori_loop(0, (nch + 1) // 2, it, 0)
    plsc.subcore_barrier()

    pltpu.sync_copy(
        accum.at[pl.ds(s * _BAND, _BAND)],
        out_hbm.at[pl.ds(c * _CORE_ROWS + s * _BAND, _BAND)])


def _neigh_call(x_pad, mc, md, cnt, zeros):
    fn = pl.kernel(
        _neigh_body,
        out_type=jax.ShapeDtypeStruct((_N_PAD, _D), jnp.float32),
        mesh=plsc.VectorSubcoreMesh(core_axis_name="c", subcore_axis_name="s"),
        scratch_types=[
            pltpu.VMEM((_CHUNK,), jnp.int32),
            pltpu.VMEM((_CHUNK,), jnp.int32),
            pltpu.VMEM((_CHUNK,), jnp.int32),
            pltpu.VMEM((_CHUNK, _D), jnp.float32),
            pltpu.VMEM((_CHUNK, _D), jnp.float32),
            pltpu.VMEM((16,), jnp.int32),
            pltpu.VMEM_SHARED((_ACC_ROWS, _D), jnp.float32),
            pltpu.SemaphoreType.DMA,
            pltpu.SemaphoreType.DMA,
        ],
        compiler_params=pltpu.CompilerParams(needs_layout_passes=False),
    )
    return fn(x_pad, mc, md, cnt, zeros)


_BLK = 1280


def _tc_layer_body(x_ref, n_ref, ws_ref, bs_ref, wn_ref, bn_ref,
                   wfp_ref, seg_ref, fp_in_ref, xo_ref, fp_ref):
    i = pl.program_id(0)
    x = x_ref[...]
    n = n_ref[...]
    h = jnp.tanh(
        lax.dot_general(x, ws_ref[...], (((1,), (1,)), ((), ())),
                        preferred_element_type=jnp.float32)
        + bs_ref[...]
        + lax.dot_general(n, wn_ref[...], (((1,), (1,)), ((), ())),
                          preferred_element_type=jnp.float32)
        + bn_ref[...])
    rows = lax.broadcasted_iota(jnp.int32, (_BLK, 1), 0) + i * _BLK
    h = jnp.where(rows < _N, h, 0.0)
    xo_ref[...] = h

    logits = lax.dot_general(h, wfp_ref[...], (((1,), (1,)), ((), ())),
                             preferred_element_type=jnp.float32)
    m = jnp.max(logits, axis=1, keepdims=True)
    e = jnp.exp(logits - m)
    p = e / jnp.sum(e, axis=1, keepdims=True)

    seg = seg_ref[...]
    onehot = (seg == lax.broadcasted_iota(jnp.int32, (_BLK, _G), 1)
              ).astype(jnp.float32)
    delta = lax.dot_general(onehot, p, (((0,), (0,)), ((), ())),
                            preferred_element_type=jnp.float32)

    @pl.when(i == 0)
    def _():
        fp_ref[...] = fp_in_ref[...] + delta

    @pl.when(i != 0)
    def _():
        fp_ref[...] = fp_ref[...] + delta


def _tc_layer(x_pad, n, ws, bs, wn, bn, wfp, seg, fp_in):
    nblk = _N_PAD // _BLK
    row_spec = pl.BlockSpec((_BLK, _D), lambda i: (i, 0))
    full_spec = pl.BlockSpec((_D, _D), lambda i: (0, 0))
    bias_spec = pl.BlockSpec((1, _D), lambda i: (0, 0))
    seg_spec = pl.BlockSpec((_BLK, 1), lambda i: (i, 0))
    fp_spec = pl.BlockSpec((_G, _D), lambda i: (0, 0))
    return pl.pallas_call(
        _tc_layer_body,
        grid=(nblk,),
        in_specs=[row_spec, row_spec, full_spec, bias_spec,
                  full_spec, bias_spec, full_spec, seg_spec, fp_spec],
        out_specs=[row_spec, fp_spec],
        out_shape=[jax.ShapeDtypeStruct((_N_PAD, _D), jnp.float32),
                   jax.ShapeDtypeStruct((_G, _D), jnp.float32)],
    )(x_pad, n, ws, bs, wn, bn, wfp, seg, fp_in)


def kernel(x, edge_index, batch, W_self_w, W_self_b, W_neigh_w, W_neigh_b,
           W_fp_w):
    x = x.astype(jnp.float32)
    row = edge_index[0].astype(jnp.int32)
    col = edge_index[1].astype(jnp.int32)

    mc, md, cnt = _prepass(row, col)

    x_pad = jnp.concatenate(
        [x, jnp.zeros((_N_PAD - _N, _D), jnp.float32)], axis=0)
    seg = jnp.concatenate(
        [batch.astype(jnp.int32),
         jnp.full((_N_PAD - _N,), _G, jnp.int32)]).reshape(_N_PAD, 1)
    zeros = jnp.zeros((_ACC_ROWS, _D), jnp.float32)

    fp = jnp.zeros((_G, _D), jnp.float32)
    for layer in range(_NUM_LAYERS):
        nb = _neigh_call(x_pad, mc, md, cnt, zeros)
        x_pad, fp = _tc_layer(
            x_pad, nb,
            W_self_w[layer], W_self_b[layer].reshape(1, _D),
            W_neigh_w[layer], W_neigh_b[layer].reshape(1, _D),
            W_fp_w[layer], seg, fp)
    return fp
